# baseline (speedup 1.0000x reference)
import functools

import jax
import jax.numpy as jnp
from jax import lax
from jax.experimental import pallas as pl
from jax.experimental.pallas import tpu as pltpu
from jax.experimental.pallas import tpu_sc as plsc

T, N, G, E = 4, 2048, 4096, 32768
D, H = 128, 256

NC, NS = 2, 16
EC = E // (NC * NS)
RS = N // NS


def _pre_body(x_ref, wn_ref, ws_ref, bg_ref, xwn_ref, xws_ref):
    x = x_ref[0]
    xwn_ref[0] = jnp.dot(x, wn_ref[...], preferred_element_type=jnp.float32)
    xws_ref[0] = jnp.dot(x, ws_ref[...], preferred_element_type=jnp.float32) + bg_ref[...]


def _gnn_pre(x, wn, ws, bg):
    return pl.pallas_call(
        _pre_body,
        grid=(T,),
        in_specs=[
            pl.BlockSpec((1, N, D), lambda t: (t, 0, 0)),
            pl.BlockSpec((D, D), lambda t: (0, 0)),
            pl.BlockSpec((D, D), lambda t: (0, 0)),
            pl.BlockSpec((1, D), lambda t: (0, 0)),
        ],
        out_specs=[
            pl.BlockSpec((1, N, D), lambda t: (t, 0, 0)),
            pl.BlockSpec((1, N, D), lambda t: (t, 0, 0)),
        ],
        out_shape=[
            jax.ShapeDtypeStruct((T, N, D), jnp.float32),
            jax.ShapeDtypeStruct((T, N, D), jnp.float32),
        ],
    )(x, wn, ws, bg)


CH = 128
KC = EC // CH


NB = 6


def _seg_body(xwn_ref, edge_ref, zeros_ref, out_ref, deg_ref,
              src_v, dst_v, rows_v, deg_v, agg_sh, gsems, ssems):
    c = lax.axis_index("c")
    s = lax.axis_index("s")
    w = c * NS + s
    ones16 = jnp.ones((16,), jnp.float32)
    zeros16 = jnp.zeros((16,), jnp.float32)
    for t in range(T):
        pltpu.sync_copy(zeros_ref.at[pl.ds(s * RS, RS)],
                        agg_sh.at[pl.ds(s * RS, RS)])
        for i in range(N // 16):
            deg_v[pl.ds(i * 16, 16)] = zeros16
        pltpu.sync_copy(edge_ref.at[t, 0, pl.ds(w * KC, KC)], src_v)
        pltpu.sync_copy(edge_ref.at[t, 1, pl.ds(w * KC, KC)], dst_v)
        for j in range(KC):
            for i in range(CH // 16):
                sl = pl.ds(i * 16, 16)
                src_v[j, sl] = src_v[j, sl] + t * N
        for j in range(KC):
            for i in range(CH // 16):
                idx = dst_v[j, pl.ds(i * 16, 16)]
                plsc.addupdate_scatter(deg_v, [idx], ones16)
        pltpu.sync_copy(deg_v, deg_ref.at[t, w])
        plsc.subcore_barrier()
        gd = [None] * KC
        sd = [None] * KC
        for j in range(NB):
            gd[j] = pltpu.async_copy(xwn_ref.at[src_v.at[j]],
                                     rows_v.at[j % NB], gsems[j % NB])
        for j in range(KC):
            b = j % NB
            gd[j].wait()
            sd[j] = pltpu.async_copy(rows_v.at[b], agg_sh.at[dst_v.at[j]],
                                     ssems[b], add=True)
            if j + NB < KC:
                sd[j].wait()
                gd[j + NB] = pltpu.async_copy(xwn_ref.at[src_v.at[j + NB]],
                                              rows_v.at[b], gsems[b])
        for j in range(KC - NB, KC):
            sd[j].wait()
        plsc.subcore_barrier()
        pltpu.sync_copy(agg_sh.at[pl.ds(s * RS, RS)],
                        out_ref.at[t, c, pl.ds(s * RS, RS)])
        plsc.subcore_barrier()


def _segment_agg(xwn_flat, edge_chunks, zeros_nd):
    return pl.kernel(
        _seg_body,
        out_type=[
            jax.ShapeDtypeStruct((T, NC, N, D), jnp.float32),
            jax.ShapeDtypeStruct((T, NC * NS, N), jnp.float32),
        ],
        mesh=plsc.VectorSubcoreMesh(core_axis_name="c", subcore_axis_name="s"),
        compiler_params=pltpu.CompilerParams(needs_layout_passes=False),
        scratch_types=[
            pltpu.VMEM((KC, CH), jnp.int32),
            pltpu.VMEM((KC, CH), jnp.int32),
            pltpu.VMEM((NB, CH, D), jnp.float32),
            pltpu.VMEM((N,), jnp.float32),
            pltpu.VMEM_SHARED((N, D), jnp.float32),
            [pltpu.SemaphoreType.DMA] * NB,
            [pltpu.SemaphoreType.DMA] * NB,
        ],
    )(xwn_flat, edge_chunks, zeros_nd)


def _comb_body(parts_ref, degs_ref, xws_ref, h_ref):
    t = pl.program_id(0)

    @pl.when(t < T)
    def _():
        tot = parts_ref[0, 0] + parts_ref[0, 1]
        ones_w = jnp.ones((NC * NS, 1), jnp.float32)
        deg = lax.dot_general(degs_ref[0], ones_w, (((0,), (0,)), ((), ())),
                              preferred_element_type=jnp.float32)
        deg = jnp.maximum(deg, 1.0)
        h_ref[0] = jax.nn.relu(xws_ref[0] + tot / deg)

    @pl.when(t == T)
    def _():
        h_ref[0] = jnp.zeros((N, D), jnp.float32)


def _combine(parts, degs, xws):
    return pl.pallas_call(
        _comb_body,
        grid=(T + 1,),
        in_specs=[
            pl.BlockSpec((1, NC, N, D), lambda t: (jnp.minimum(t, T - 1), 0, 0, 0)),
            pl.BlockSpec((1, NC * NS, N), lambda t: (jnp.minimum(t, T - 1), 0, 0)),
            pl.BlockSpec((1, N, D), lambda t: (jnp.minimum(t, T - 1), 0, 0)),
        ],
        out_specs=pl.BlockSpec((1, N, D), lambda t: (t, 0, 0)),
        out_shape=jax.ShapeDtypeStruct((T + 1, N, D), jnp.float32),
    )(parts, degs, xws)


ZROW = T * N
GR = 2 * G // NS


def _align_body(h_ref, l2g_ref, out_ref, lg_v, w_v, idx_v, row_v, wsh_ref, gsem):
    c = lax.axis_index("c")
    s = lax.axis_index("s")
    iota16 = lax.iota(jnp.int32, 16)

    @pl.when(s < 2)
    def _():
        t = c * 2 + s
        pltpu.sync_copy(l2g_ref.at[t], lg_v)

        def init_u(u, carry):
            w_v[pl.ds(u * 16, 16)] = jnp.full((16,), -1, jnp.int32)
            return carry
        lax.fori_loop(0, G // 16, init_u, 0)

        def pass_j(j, ch):
            i16 = j * 16 + iota16
            g16 = lg_v[pl.ds(j * 16, 16)]
            rb = plsc.load_gather(w_v, [g16])
            m = rb < i16
            plsc.store_scatter(w_v, [g16], i16, mask=m)
            return jnp.maximum(ch, jnp.max(jnp.where(m, 1, 0)))

        def fp_cond(carry):
            return (carry[0] > 0) & (carry[1] < 24)

        def fp_body(carry):
            ch = lax.fori_loop(0, N // 16, pass_j, jnp.int32(0))
            return (ch, carry[1] + 1)

        lax.while_loop(fp_cond, fp_body, (jnp.int32(1), jnp.int32(0)))

        def flat_u(u, carry):
            sl = pl.ds(u * 16, 16)
            v = w_v[sl]
            w_v[sl] = jnp.where(v < 0, ZROW, v + t * N)
            return carry
        lax.fori_loop(0, G // 16, flat_u, 0)
        pltpu.sync_copy(w_v, wsh_ref.at[s])

    plsc.subcore_barrier()
    for k in range(GR // CH):
        r = s * GR + k * CH
        tl, g0 = r // G, r % G
        pltpu.sync_copy(wsh_ref.at[tl, pl.ds(g0, CH)], idx_v.at[k])
        pltpu.async_copy(h_ref.at[idx_v.at[k]], row_v, gsem).wait()
        pltpu.sync_copy(row_v, out_ref.at[pl.ds(c * 2 * G + r, CH)])


def _align(h_pad_flat, l2g_i32):
    return pl.kernel(
        _align_body,
        out_type=jax.ShapeDtypeStruct((T * G, D), jnp.float32),
        mesh=plsc.VectorSubcoreMesh(core_axis_name="c", subcore_axis_name="s"),
        compiler_params=pltpu.CompilerParams(needs_layout_passes=False),
        scratch_types=[
            pltpu.VMEM((N,), jnp.int32),
            pltpu.VMEM((G,), jnp.int32),
            pltpu.VMEM((GR // CH, CH), jnp.int32),
            pltpu.VMEM((CH, D), jnp.float32),
            pltpu.VMEM_SHARED((2, G), jnp.int32),
            pltpu.SemaphoreType.DMA,
        ],
    )(h_pad_flat, l2g_i32)


BG = 1024


def _lstm_body(ti_ref, al_ref, wt_ref, bt_ref, wih_ref, whh_ref, bl_ref,
               wc1_ref, bc1_ref, wc2_ref, bc2_ref, out_ref):
    h = jnp.zeros((BG, H), jnp.float32)
    c = jnp.zeros((BG, H), jnp.float32)
    for t in range(T):
        tg = jax.nn.sigmoid(ti_ref[t, 0] * wt_ref[...] + bt_ref[...])
        gates = (jnp.dot(al_ref[t], wih_ref[...], preferred_element_type=jnp.float32)
                 + jnp.dot(h, whh_ref[...], preferred_element_type=jnp.float32)
                 + bl_ref[...])
        i_g = gates[:, 0 * H:1 * H]
        f_g = gates[:, 1 * H:2 * H]
        g_g = gates[:, 2 * H:3 * H]
        o_g = gates[:, 3 * H:4 * H]
        c = jax.nn.sigmoid(f_g) * (c * tg) + jax.nn.sigmoid(i_g) * jnp.tanh(g_g)
        h = jax.nn.sigmoid(o_g) * jnp.tanh(c)
    hid = jax.nn.relu(jnp.dot(h, wc1_ref[...], preferred_element_type=jnp.float32)
                      + bc1_ref[...])
    out_ref[...] = (jnp.dot(hid, wc2_ref[...], preferred_element_type=jnp.float32)
                    + bc2_ref[...])


def _lstm_classify(ti, aligned, wt, bt, wih, whh, bl, wc1, bc1, wc2p, bc2p):
    return pl.pallas_call(
        _lstm_body,
        grid=(G // BG,),
        in_specs=[
            pl.BlockSpec(memory_space=pltpu.SMEM),
            pl.BlockSpec((T, BG, D), lambda i: (0, i, 0)),
            pl.BlockSpec((1, H), lambda i: (0, 0)),
            pl.BlockSpec((1, H), lambda i: (0, 0)),
            pl.BlockSpec((D, 4 * H), lambda i: (0, 0)),
            pl.BlockSpec((H, 4 * H), lambda i: (0, 0)),
            pl.BlockSpec((1, 4 * H), lambda i: (0, 0)),
            pl.BlockSpec((H, H // 2), lambda i: (0, 0)),
            pl.BlockSpec((1, H // 2), lambda i: (0, 0)),
            pl.BlockSpec((H // 2, 128), lambda i: (0, 0)),
            pl.BlockSpec((1, 128), lambda i: (0, 0)),
        ],
        out_specs=pl.BlockSpec((BG, 128), lambda i: (i, 0)),
        out_shape=jax.ShapeDtypeStruct((G, 128), jnp.float32),
    )(ti, aligned, wt, bt, wih, whh, bl, wc1, bc1, wc2p, bc2p)


def kernel(time_intervals, x, edge_index, local_to_global,
           W_gnn_self, W_gnn_nbr, b_gnn,
           W_ih, W_hh, b_lstm, W_time, b_time,
           W_c1, b_c1, W_c2, b_c2):
    edge_i32 = edge_index.astype(jnp.int32).reshape(T, 2, E // CH, CH)
    l2g = local_to_global.astype(jnp.int32)

    xwn, xws = _gnn_pre(x, W_gnn_nbr, W_gnn_self, b_gnn.reshape(1, D))

    zeros_nd = jnp.zeros((N, D), jnp.float32)
    parts, degs = _segment_agg(xwn.reshape(T * N, D), edge_i32, zeros_nd)
    h_pad = _combine(parts, degs, xws)
    aligned = _align(h_pad.reshape((T + 1) * N, D), l2g).reshape(T, G, D)

    ti = time_intervals.reshape(T, 1)
    wc2p = jnp.zeros((H // 2, 128), jnp.float32).at[:, :3].set(W_c2)
    bc2p = jnp.zeros((1, 128), jnp.float32).at[0, :3].set(b_c2)
    out = _lstm_classify(ti, aligned, W_time.reshape(1, H), b_time.reshape(1, H),
                         W_ih, W_hh, b_lstm.reshape(1, 4 * H),
                         W_c1, b_c1.reshape(1, H // 2), wc2p, bc2p)
    return out[:, :3]


# speedup vs baseline: 1.0030x; 1.0030x over previous
import functools

import jax
import jax.numpy as jnp
from jax import lax
from jax.experimental import pallas as pl
from jax.experimental.pallas import tpu as pltpu
from jax.experimental.pallas import tpu_sc as plsc

T, N, G, E = 4, 2048, 4096, 32768
D, H = 128, 256

NC, NS = 2, 16
EC = E // (NC * NS)
RS = N // NS


def _pre_body(x_ref, wn_ref, ws_ref, bg_ref, xwn_ref, xws_ref):
    x = x_ref[0]
    xwn_ref[0] = jnp.dot(x, wn_ref[...], preferred_element_type=jnp.float32)
    xws_ref[0] = jnp.dot(x, ws_ref[...], preferred_element_type=jnp.float32) + bg_ref[...]


def _gnn_pre(x, wn, ws, bg):
    return pl.pallas_call(
        _pre_body,
        grid=(T,),
        in_specs=[
            pl.BlockSpec((1, N, D), lambda t: (t, 0, 0)),
            pl.BlockSpec((D, D), lambda t: (0, 0)),
            pl.BlockSpec((D, D), lambda t: (0, 0)),
            pl.BlockSpec((1, D), lambda t: (0, 0)),
        ],
        out_specs=[
            pl.BlockSpec((1, N, D), lambda t: (t, 0, 0)),
            pl.BlockSpec((1, N, D), lambda t: (t, 0, 0)),
        ],
        out_shape=[
            jax.ShapeDtypeStruct((T, N, D), jnp.float32),
            jax.ShapeDtypeStruct((T, N, D), jnp.float32),
        ],
    )(x, wn, ws, bg)


CH = 128
KC = EC // CH


NB = 4


def _seg_body(xwn_ref, edge_ref, zeros_ref, l2g_ref, negones_ref,
              out_ref, deg_ref, w_ref,
              src_v, dst_v, rows_v, deg_v, lg_v, w_v, agg_sh, gsems, ssems):
    c = lax.axis_index("c")
    s = lax.axis_index("s")
    w = c * NS + s
    ones16 = jnp.ones((16,), jnp.float32)
    zeros16 = jnp.zeros((16,), jnp.float32)
    iota16 = lax.iota(jnp.int32, 16)

    @pl.when(s < 2)
    def _():
        tb = c * 2 + s
        pltpu.sync_copy(l2g_ref.at[pl.ds(tb * N, N)], lg_v)
        pltpu.sync_copy(negones_ref, w_v)

        def pass_j(j, ch):
            i16 = j * 16 + iota16
            g16 = lg_v[pl.ds(j * 16, 16)]
            rb = plsc.load_gather(w_v, [g16])
            m = rb < i16
            plsc.store_scatter(w_v, [g16], i16, mask=m)
            return jnp.maximum(ch, jnp.max(jnp.where(m, 1, 0)))

        def fp_cond(carry):
            return (carry[0] > 0) & (carry[1] < 64)

        def fp_body(carry):
            ch = lax.fori_loop(0, N // 16, pass_j, jnp.int32(0))
            return (ch, carry[1] + 1)

        lax.while_loop(fp_cond, fp_body, (jnp.int32(1), jnp.int32(0)))
        pltpu.sync_copy(w_v, w_ref.at[pl.ds(tb * G, G)])

    for t in range(T):
        pltpu.sync_copy(zeros_ref.at[pl.ds(s * RS, RS)],
                        agg_sh.at[pl.ds(s * RS, RS)])
        for i in range(N // 16):
            deg_v[pl.ds(i * 16, 16)] = zeros16
        pltpu.sync_copy(edge_ref.at[t, 0, pl.ds(w * KC, KC)], src_v)
        pltpu.sync_copy(edge_ref.at[t, 1, pl.ds(w * KC, KC)], dst_v)
        for j in range(KC):
            for i in range(CH // 16):
                sl = pl.ds(i * 16, 16)
                src_v[j, sl] = src_v[j, sl] + t * N
        for j in range(KC):
            for i in range(CH // 16):
                idx = dst_v[j, pl.ds(i * 16, 16)]
                plsc.addupdate_scatter(deg_v, [idx], ones16)
        pltpu.sync_copy(deg_v, deg_ref.at[t, w])
        plsc.subcore_barrier()
        gd = [None] * KC
        sd = [None] * KC
        for j in range(NB):
            gd[j] = pltpu.async_copy(xwn_ref.at[src_v.at[j]],
                                     rows_v.at[j % NB], gsems[j % NB])
        for j in range(KC):
            b = j % NB
            gd[j].wait()
            sd[j] = pltpu.async_copy(rows_v.at[b], agg_sh.at[dst_v.at[j]],
                                     ssems[b], add=True)
            if j + NB < KC:
                sd[j].wait()
                gd[j + NB] = pltpu.async_copy(xwn_ref.at[src_v.at[j + NB]],
                                              rows_v.at[b], gsems[b])
        for j in range(KC - NB, KC):
            sd[j].wait()
        plsc.subcore_barrier()
        pltpu.sync_copy(agg_sh.at[pl.ds(s * RS, RS)],
                        out_ref.at[t, c, pl.ds(s * RS, RS)])
        plsc.subcore_barrier()


def _segment_agg(xwn_flat, edge_chunks, zeros_nd, l2g_flat, negones):
    return pl.kernel(
        _seg_body,
        out_type=[
            jax.ShapeDtypeStruct((T, NC, N, D), jnp.float32),
            jax.ShapeDtypeStruct((T, NC * NS, N), jnp.float32),
            jax.ShapeDtypeStruct((T * G,), jnp.int32),
        ],
        mesh=plsc.VectorSubcoreMesh(core_axis_name="c", subcore_axis_name="s"),
        compiler_params=pltpu.CompilerParams(needs_layout_passes=False),
        scratch_types=[
            pltpu.VMEM((KC, CH), jnp.int32),
            pltpu.VMEM((KC, CH), jnp.int32),
            pltpu.VMEM((NB, CH, D), jnp.float32),
            pltpu.VMEM((N,), jnp.float32),
            pltpu.VMEM((N,), jnp.int32),
            pltpu.VMEM((G,), jnp.int32),
            pltpu.VMEM_SHARED((N, D), jnp.float32),
            [pltpu.SemaphoreType.DMA] * NB,
            [pltpu.SemaphoreType.DMA] * NB,
        ],
    )(xwn_flat, edge_chunks, zeros_nd, l2g_flat, negones)


def _comb_body(parts_ref, degs_ref, xws_ref, h_ref):
    t = pl.program_id(0)

    @pl.when(t < T)
    def _():
        tot = parts_ref[0, 0] + parts_ref[0, 1]
        ones_w = jnp.ones((NC * NS, 1), jnp.float32)
        deg = lax.dot_general(degs_ref[0], ones_w, (((0,), (0,)), ((), ())),
                              preferred_element_type=jnp.float32)
        deg = jnp.maximum(deg, 1.0)
        h_ref[0] = jax.nn.relu(xws_ref[0] + tot / deg)

    @pl.when(t == T)
    def _():
        h_ref[0] = jnp.zeros((N, D), jnp.float32)


def _combine(parts, degs, xws):
    return pl.pallas_call(
        _comb_body,
        grid=(T + 1,),
        in_specs=[
            pl.BlockSpec((1, NC, N, D), lambda t: (jnp.minimum(t, T - 1), 0, 0, 0)),
            pl.BlockSpec((1, NC * NS, N), lambda t: (jnp.minimum(t, T - 1), 0, 0)),
            pl.BlockSpec((1, N, D), lambda t: (jnp.minimum(t, T - 1), 0, 0)),
        ],
        out_specs=pl.BlockSpec((1, N, D), lambda t: (t, 0, 0)),
        out_shape=jax.ShapeDtypeStruct((T + 1, N, D), jnp.float32),
    )(parts, degs, xws)


ZROW = T * N
GR = 2 * G // NS
ND = 2


def _align_body(h_ref, w_ref, out_ref, idx_v, row_v, gsems, osems):
    c = lax.axis_index("c")
    s = lax.axis_index("s")
    KD = GR // CH
    for k in range(KD):
        r = s * GR + k * CH
        pltpu.sync_copy(w_ref.at[pl.ds(c * 2 * G + r, CH)], idx_v.at[k])
    for k in range(KD):
        tl = (s * GR + k * CH) // G
        toff = (c * 2 + tl) * N
        for i in range(CH // 16):
            sl = pl.ds(i * 16, 16)
            v = idx_v[k, sl]
            idx_v[k, sl] = jnp.where(v < 0, ZROW, v + toff)
    gd = [None] * KD
    od = [None] * KD
    for k in range(KD):
        b = k % ND
        if k >= ND:
            od[k - ND].wait()
        gd[k] = pltpu.async_copy(h_ref.at[idx_v.at[k]], row_v.at[b], gsems[b])
        gd[k].wait()
        od[k] = pltpu.async_copy(
            row_v.at[b], out_ref.at[pl.ds(c * 2 * G + s * GR + k * CH, CH)],
            osems[b])
    for k in range(KD - ND, KD):
        od[k].wait()


def _align(h_pad_flat, w_flat):
    return pl.kernel(
        _align_body,
        out_type=jax.ShapeDtypeStruct((T * G, D), jnp.float32),
        mesh=plsc.VectorSubcoreMesh(core_axis_name="c", subcore_axis_name="s"),
        compiler_params=pltpu.CompilerParams(needs_layout_passes=False),
        scratch_types=[
            pltpu.VMEM((GR // CH, CH), jnp.int32),
            pltpu.VMEM((ND, CH, D), jnp.float32),
            [pltpu.SemaphoreType.DMA] * ND,
            [pltpu.SemaphoreType.DMA] * ND,
        ],
    )(h_pad_flat, w_flat)


BG = 1024


def _lstm_body(ti_ref, al_ref, wt_ref, bt_ref, wih_ref, whh_ref, bl_ref,
               wc1_ref, bc1_ref, wc2_ref, bc2_ref, out_ref):
    h = jnp.zeros((BG, H), jnp.float32)
    c = jnp.zeros((BG, H), jnp.float32)
    for t in range(T):
        tg = jax.nn.sigmoid(ti_ref[t, 0] * wt_ref[...] + bt_ref[...])
        gates = (jnp.dot(al_ref[t], wih_ref[...], preferred_element_type=jnp.float32)
                 + jnp.dot(h, whh_ref[...], preferred_element_type=jnp.float32)
                 + bl_ref[...])
        i_g = gates[:, 0 * H:1 * H]
        f_g = gates[:, 1 * H:2 * H]
        g_g = gates[:, 2 * H:3 * H]
        o_g = gates[:, 3 * H:4 * H]
        c = jax.nn.sigmoid(f_g) * (c * tg) + jax.nn.sigmoid(i_g) * jnp.tanh(g_g)
        h = jax.nn.sigmoid(o_g) * jnp.tanh(c)
    hid = jax.nn.relu(jnp.dot(h, wc1_ref[...], preferred_element_type=jnp.float32)
                      + bc1_ref[...])
    out_ref[...] = (jnp.dot(hid, wc2_ref[...], preferred_element_type=jnp.float32)
                    + bc2_ref[...])


def _lstm_classify(ti, aligned, wt, bt, wih, whh, bl, wc1, bc1, wc2p, bc2p):
    return pl.pallas_call(
        _lstm_body,
        grid=(G // BG,),
        in_specs=[
            pl.BlockSpec(memory_space=pltpu.SMEM),
            pl.BlockSpec((T, BG, D), lambda i: (0, i, 0)),
            pl.BlockSpec((1, H), lambda i: (0, 0)),
            pl.BlockSpec((1, H), lambda i: (0, 0)),
            pl.BlockSpec((D, 4 * H), lambda i: (0, 0)),
            pl.BlockSpec((H, 4 * H), lambda i: (0, 0)),
            pl.BlockSpec((1, 4 * H), lambda i: (0, 0)),
            pl.BlockSpec((H, H // 2), lambda i: (0, 0)),
            pl.BlockSpec((1, H // 2), lambda i: (0, 0)),
            pl.BlockSpec((H // 2, 128), lambda i: (0, 0)),
            pl.BlockSpec((1, 128), lambda i: (0, 0)),
        ],
        out_specs=pl.BlockSpec((BG, 128), lambda i: (i, 0)),
        out_shape=jax.ShapeDtypeStruct((G, 128), jnp.float32),
    )(ti, aligned, wt, bt, wih, whh, bl, wc1, bc1, wc2p, bc2p)


def kernel(time_intervals, x, edge_index, local_to_global,
           W_gnn_self, W_gnn_nbr, b_gnn,
           W_ih, W_hh, b_lstm, W_time, b_time,
           W_c1, b_c1, W_c2, b_c2):
    edge_i32 = edge_index.astype(jnp.int32).reshape(T, 2, E // CH, CH)
    l2g = local_to_global.astype(jnp.int32)

    xwn, xws = _gnn_pre(x, W_gnn_nbr, W_gnn_self, b_gnn.reshape(1, D))

    zeros_nd = jnp.zeros((N, D), jnp.float32)
    negones = jnp.full((G,), -1, jnp.int32)
    parts, degs, w_flat = _segment_agg(xwn.reshape(T * N, D), edge_i32,
                                       zeros_nd, l2g.reshape(T * N), negones)
    h_pad = _combine(parts, degs, xws)
    aligned = _align(h_pad.reshape((T + 1) * N, D), w_flat).reshape(T, G, D)

    ti = time_intervals.reshape(T, 1)
    wc2p = jnp.zeros((H // 2, 128), jnp.float32).at[:, :3].set(W_c2)
    bc2p = jnp.zeros((1, 128), jnp.float32).at[0, :3].set(b_c2)
    out = _lstm_classify(ti, aligned, W_time.reshape(1, H), b_time.reshape(1, H),
                         W_ih, W_hh, b_lstm.reshape(1, 4 * H),
                         W_c1, b_c1.reshape(1, H // 2), wc2p, bc2p)
    return out[:, :3]


# speedup vs baseline: 1.0097x; 1.0067x over previous
import functools

import jax
import jax.numpy as jnp
from jax import lax
from jax.experimental import pallas as pl
from jax.experimental.pallas import tpu as pltpu
from jax.experimental.pallas import tpu_sc as plsc

T, N, G, E = 4, 2048, 4096, 32768
D, H = 128, 256

NC, NS = 2, 16
EC = E // (NC * NS)
RS = N // NS


def _pre_body(x_ref, wn_ref, ws_ref, bg_ref, xwn_ref, xws_ref):
    x = x_ref[0]
    xwn_ref[0] = jnp.dot(x, wn_ref[...], preferred_element_type=jnp.float32)
    xws_ref[0] = jnp.dot(x, ws_ref[...], preferred_element_type=jnp.float32) + bg_ref[...]


def _gnn_pre(x, wn, ws, bg):
    return pl.pallas_call(
        _pre_body,
        grid=(T,),
        in_specs=[
            pl.BlockSpec((1, N, D), lambda t: (t, 0, 0)),
            pl.BlockSpec((D, D), lambda t: (0, 0)),
            pl.BlockSpec((D, D), lambda t: (0, 0)),
            pl.BlockSpec((1, D), lambda t: (0, 0)),
        ],
        out_specs=[
            pl.BlockSpec((1, N, D), lambda t: (t, 0, 0)),
            pl.BlockSpec((1, N, D), lambda t: (t, 0, 0)),
        ],
        out_shape=[
            jax.ShapeDtypeStruct((T, N, D), jnp.float32),
            jax.ShapeDtypeStruct((T, N, D), jnp.float32),
        ],
    )(x, wn, ws, bg)


CH = 128
KC = EC // CH


NB = 4


def _seg_body(xwn_ref, edge_ref, zeros_ref, l2g_ref, negones_ref,
              out_ref, deg_ref, w_ref,
              src_v, dst_v, rows_v, deg_v, lg_v, w_v, agg_sh, gsems, ssems):
    c = lax.axis_index("c")
    s = lax.axis_index("s")
    w = c * NS + s
    ones16 = jnp.ones((16,), jnp.float32)
    zeros16 = jnp.zeros((16,), jnp.float32)
    iota16 = lax.iota(jnp.int32, 16)

    @pl.when(s < 2)
    def _():
        tb = c * 2 + s
        pltpu.sync_copy(l2g_ref.at[pl.ds(tb * N, N)], lg_v)
        pltpu.sync_copy(negones_ref, w_v)

        def pass_j(j, ch):
            i16 = j * 16 + iota16
            g16 = lg_v[pl.ds(j * 16, 16)]
            rb = plsc.load_gather(w_v, [g16])
            m = rb < i16
            plsc.store_scatter(w_v, [g16], i16, mask=m)
            return jnp.maximum(ch, jnp.max(jnp.where(m, 1, 0)))

        def fp_cond(carry):
            return (carry[0] > 0) & (carry[1] < 64)

        def fp_body(carry):
            ch = lax.fori_loop(0, N // 16, pass_j, jnp.int32(0))
            return (ch, carry[1] + 1)

        lax.while_loop(fp_cond, fp_body, (jnp.int32(1), jnp.int32(0)))
        pltpu.sync_copy(w_v, w_ref.at[pl.ds(tb * G, G)])

    for t in range(T):
        pltpu.sync_copy(zeros_ref.at[pl.ds(s * RS, RS)],
                        agg_sh.at[pl.ds(s * RS, RS)])
        for i in range(N // 16):
            deg_v[pl.ds(i * 16, 16)] = zeros16
        pltpu.sync_copy(edge_ref.at[t, 0, pl.ds(w * KC, KC)], src_v)
        pltpu.sync_copy(edge_ref.at[t, 1, pl.ds(w * KC, KC)], dst_v)
        for j in range(KC):
            for i in range(CH // 16):
                sl = pl.ds(i * 16, 16)
                src_v[j, sl] = src_v[j, sl] + t * N
        for j in range(KC):
            for i in range(CH // 16):
                idx = dst_v[j, pl.ds(i * 16, 16)]
                plsc.addupdate_scatter(deg_v, [idx], ones16)
        pltpu.sync_copy(deg_v, deg_ref.at[t, w])
        plsc.subcore_barrier()
        gd = [None] * KC
        sd = [None] * KC
        for j in range(NB):
            gd[j] = pltpu.async_copy(xwn_ref.at[src_v.at[j]],
                                     rows_v.at[j % NB], gsems[j % NB])
        for j in range(KC):
            b = j % NB
            gd[j].wait()
            sd[j] = pltpu.async_copy(rows_v.at[b], agg_sh.at[dst_v.at[j]],
                                     ssems[b], add=True)
            if j + NB < KC:
                sd[j].wait()
                gd[j + NB] = pltpu.async_copy(xwn_ref.at[src_v.at[j + NB]],
                                              rows_v.at[b], gsems[b])
        for j in range(KC - NB, KC):
            sd[j].wait()
        plsc.subcore_barrier()
        pltpu.sync_copy(agg_sh.at[pl.ds(s * RS, RS)],
                        out_ref.at[t, c, pl.ds(s * RS, RS)])
        plsc.subcore_barrier()


def _segment_agg(xwn_flat, edge_chunks, zeros_nd, l2g_flat, negones):
    return pl.kernel(
        _seg_body,
        out_type=[
            jax.ShapeDtypeStruct((T, NC, N, D), jnp.float32),
            jax.ShapeDtypeStruct((T, NC * NS, N), jnp.float32),
            jax.ShapeDtypeStruct((T * G,), jnp.int32),
        ],
        mesh=plsc.VectorSubcoreMesh(core_axis_name="c", subcore_axis_name="s"),
        compiler_params=pltpu.CompilerParams(needs_layout_passes=False),
        scratch_types=[
            pltpu.VMEM((KC, CH), jnp.int32),
            pltpu.VMEM((KC, CH), jnp.int32),
            pltpu.VMEM((NB, CH, D), jnp.float32),
            pltpu.VMEM((N,), jnp.float32),
            pltpu.VMEM((N,), jnp.int32),
            pltpu.VMEM((G,), jnp.int32),
            pltpu.VMEM_SHARED((N, D), jnp.float32),
            [pltpu.SemaphoreType.DMA] * NB,
            [pltpu.SemaphoreType.DMA] * NB,
        ],
    )(xwn_flat, edge_chunks, zeros_nd, l2g_flat, negones)


def _comb_body(parts_ref, degs_ref, xws_ref, h_ref):
    t = pl.program_id(0)

    @pl.when(t < T)
    def _():
        tot = parts_ref[0, 0] + parts_ref[0, 1]
        ones_w = jnp.ones((NC * NS, 1), jnp.float32)
        deg = lax.dot_general(degs_ref[0], ones_w, (((0,), (0,)), ((), ())),
                              preferred_element_type=jnp.float32)
        deg = jnp.maximum(deg, 1.0)
        h_ref[0] = jax.nn.relu(xws_ref[0] + tot / deg)

    @pl.when(t == T)
    def _():
        h_ref[0] = jnp.zeros((N, D), jnp.float32)


def _combine(parts, degs, xws):
    return pl.pallas_call(
        _comb_body,
        grid=(T + 1,),
        in_specs=[
            pl.BlockSpec((1, NC, N, D), lambda t: (jnp.minimum(t, T - 1), 0, 0, 0)),
            pl.BlockSpec((1, NC * NS, N), lambda t: (jnp.minimum(t, T - 1), 0, 0)),
            pl.BlockSpec((1, N, D), lambda t: (jnp.minimum(t, T - 1), 0, 0)),
        ],
        out_specs=pl.BlockSpec((1, N, D), lambda t: (t, 0, 0)),
        out_shape=jax.ShapeDtypeStruct((T + 1, N, D), jnp.float32),
    )(parts, degs, xws)


ZROW = T * N
GR = 2 * G // NS
ND = 4


def _align_body(h_ref, w_ref, out_ref, idx_v, row_v, gsems, osems):
    c = lax.axis_index("c")
    s = lax.axis_index("s")
    KD = GR // CH
    for k in range(KD):
        r = s * GR + k * CH
        pltpu.sync_copy(w_ref.at[pl.ds(c * 2 * G + r, CH)], idx_v.at[k])
    for k in range(KD):
        tl = (s * GR + k * CH) // G
        toff = (c * 2 + tl) * N
        for i in range(CH // 16):
            sl = pl.ds(i * 16, 16)
            v = idx_v[k, sl]
            idx_v[k, sl] = jnp.where(v < 0, ZROW, v + toff)
    gd = [None] * KD
    od = [None] * KD
    for k in range(KD):
        gd[k] = pltpu.async_copy(h_ref.at[idx_v.at[k]], row_v.at[k], gsems[k])
    for k in range(KD):
        gd[k].wait()
        od[k] = pltpu.async_copy(
            row_v.at[k], out_ref.at[pl.ds(c * 2 * G + s * GR + k * CH, CH)],
            osems[k])
    for k in range(KD):
        od[k].wait()


def _align(h_pad_flat, w_flat):
    return pl.kernel(
        _align_body,
        out_type=jax.ShapeDtypeStruct((T * G, D), jnp.float32),
        mesh=plsc.VectorSubcoreMesh(core_axis_name="c", subcore_axis_name="s"),
        compiler_params=pltpu.CompilerParams(needs_layout_passes=False),
        scratch_types=[
            pltpu.VMEM((GR // CH, CH), jnp.int32),
            pltpu.VMEM((ND, CH, D), jnp.float32),
            [pltpu.SemaphoreType.DMA] * ND,
            [pltpu.SemaphoreType.DMA] * ND,
        ],
    )(h_pad_flat, w_flat)


BG = 1024


def _lstm_body(ti_ref, al_ref, wt_ref, bt_ref, wih_ref, whh_ref, bl_ref,
               wc1_ref, bc1_ref, wc2_ref, bc2_ref, out_ref):
    h = jnp.zeros((BG, H), jnp.float32)
    c = jnp.zeros((BG, H), jnp.float32)
    for t in range(T):
        tg = jax.nn.sigmoid(ti_ref[t, 0] * wt_ref[...] + bt_ref[...])
        gates = (jnp.dot(al_ref[t], wih_ref[...], preferred_element_type=jnp.float32)
                 + jnp.dot(h, whh_ref[...], preferred_element_type=jnp.float32)
                 + bl_ref[...])
        i_g = gates[:, 0 * H:1 * H]
        f_g = gates[:, 1 * H:2 * H]
        g_g = gates[:, 2 * H:3 * H]
        o_g = gates[:, 3 * H:4 * H]
        c = jax.nn.sigmoid(f_g) * (c * tg) + jax.nn.sigmoid(i_g) * jnp.tanh(g_g)
        h = jax.nn.sigmoid(o_g) * jnp.tanh(c)
    hid = jax.nn.relu(jnp.dot(h, wc1_ref[...], preferred_element_type=jnp.float32)
                      + bc1_ref[...])
    out_ref[...] = (jnp.dot(hid, wc2_ref[...], preferred_element_type=jnp.float32)
                    + bc2_ref[...])


def _lstm_classify(ti, aligned, wt, bt, wih, whh, bl, wc1, bc1, wc2p, bc2p):
    return pl.pallas_call(
        _lstm_body,
        grid=(G // BG,),
        in_specs=[
            pl.BlockSpec(memory_space=pltpu.SMEM),
            pl.BlockSpec((T, BG, D), lambda i: (0, i, 0)),
            pl.BlockSpec((1, H), lambda i: (0, 0)),
            pl.BlockSpec((1, H), lambda i: (0, 0)),
            pl.BlockSpec((D, 4 * H), lambda i: (0, 0)),
            pl.BlockSpec((H, 4 * H), lambda i: (0, 0)),
            pl.BlockSpec((1, 4 * H), lambda i: (0, 0)),
            pl.BlockSpec((H, H // 2), lambda i: (0, 0)),
            pl.BlockSpec((1, H // 2), lambda i: (0, 0)),
            pl.BlockSpec((H // 2, 128), lambda i: (0, 0)),
            pl.BlockSpec((1, 128), lambda i: (0, 0)),
        ],
        out_specs=pl.BlockSpec((BG, 128), lambda i: (i, 0)),
        out_shape=jax.ShapeDtypeStruct((G, 128), jnp.float32),
    )(ti, aligned, wt, bt, wih, whh, bl, wc1, bc1, wc2p, bc2p)


def kernel(time_intervals, x, edge_index, local_to_global,
           W_gnn_self, W_gnn_nbr, b_gnn,
           W_ih, W_hh, b_lstm, W_time, b_time,
           W_c1, b_c1, W_c2, b_c2):
    edge_i32 = edge_index.astype(jnp.int32).reshape(T, 2, E // CH, CH)
    l2g = local_to_global.astype(jnp.int32)

    xwn, xws = _gnn_pre(x, W_gnn_nbr, W_gnn_self, b_gnn.reshape(1, D))

    zeros_nd = jnp.zeros((N, D), jnp.float32)
    negones = jnp.full((G,), -1, jnp.int32)
    parts, degs, w_flat = _segment_agg(xwn.reshape(T * N, D), edge_i32,
                                       zeros_nd, l2g.reshape(T * N), negones)
    h_pad = _combine(parts, degs, xws)
    aligned = _align(h_pad.reshape((T + 1) * N, D), w_flat).reshape(T, G, D)

    ti = time_intervals.reshape(T, 1)
    wc2p = jnp.zeros((H // 2, 128), jnp.float32).at[:, :3].set(W_c2)
    bc2p = jnp.zeros((1, 128), jnp.float32).at[0, :3].set(b_c2)
    out = _lstm_classify(ti, aligned, W_time.reshape(1, H), b_time.reshape(1, H),
                         W_ih, W_hh, b_lstm.reshape(1, 4 * H),
                         W_c1, b_c1.reshape(1, H // 2), wc2p, bc2p)
    return out[:, :3]
